# baseline (speedup 1.0000x reference)
import functools

import jax
import jax.numpy as jnp
from jax import lax
from jax.experimental import pallas as pl
from jax.experimental.pallas import tpu as pltpu
from jax.experimental.pallas import tpu_sc as plsc

BATCH = 16384
EMBED = 64
NUM_CORES = 2
NUM_SUBCORES = 16
NUM_WORKERS = NUM_CORES * NUM_SUBCORES
BPW = BATCH // NUM_WORKERS
CHUNK = 128
NCHUNK = BPW // CHUNK
LANES = 16
GROUPS = BPW // LANES

_mesh = plsc.VectorSubcoreMesh(core_axis_name="c", subcore_axis_name="s")

_GATHER_DNUMS = lax.GatherDimensionNumbers(
    offset_dims=(), collapsed_slice_dims=(0,), start_index_map=(0,))


def _permute(x, pm):
    return lax.gather(x, pm[:, None], _GATHER_DNUMS, (1,),
                      mode=lax.GatherScatterMode.PROMISE_IN_BOUNDS)


@functools.partial(
    pl.kernel,
    mesh=_mesh,
    compiler_params=pltpu.CompilerParams(use_tc_tiling_on_sc=False),
    out_type=jax.ShapeDtypeStruct((BATCH,), jnp.float32),
    scratch_types=[
        pltpu.VMEM((BPW,), jnp.int32),
        pltpu.VMEM((BPW,), jnp.int32),
        pltpu.VMEM((BPW, EMBED), jnp.float32),
        pltpu.VMEM((BPW, EMBED), jnp.float32),
        pltpu.VMEM((BPW,), jnp.float32),
        pltpu.SemaphoreType.DMA,
        pltpu.SemaphoreType.DMA,
    ],
)
def _cf_dot_kernel(uid_hbm, iid_hbm, utab_hbm, itab_hbm, out_hbm,
                   uidx_v, iidx_v, urows_v, irows_v, out_v, usem, isem):
    wid = lax.axis_index("s") * NUM_CORES + lax.axis_index("c")
    base = wid * BPW

    pltpu.sync_copy(uid_hbm.at[pl.ds(base, BPW)], uidx_v)
    pltpu.sync_copy(iid_hbm.at[pl.ds(base, BPW)], iidx_v)

    copies = []
    for j in range(NCHUNK):
        sl = pl.ds(j * CHUNK, CHUNK)
        copies.append(
            pltpu.async_copy(utab_hbm.at[uidx_v.at[sl]], urows_v.at[sl], usem))
        copies.append(
            pltpu.async_copy(itab_hbm.at[iidx_v.at[sl]], irows_v.at[sl], isem))
    for c in copies:
        c.wait()

    lane_iota = lax.iota(jnp.int32, LANES)
    perms = [(lane_iota + (1 << k)) & (LANES - 1) for k in range(4)]

    def group_body(g, carry):
        acc = jnp.zeros((LANES,), jnp.float32)
        for r in range(LANES):
            b = g * LANES + r
            p = jnp.zeros((LANES,), jnp.float32)
            for k in range(EMBED // LANES):
                sl = pl.ds(k * LANES, LANES)
                p = p + urows_v[b, sl] * irows_v[b, sl]
            for pm in perms:
                p = p + _permute(p, pm)
            acc = jnp.where(lane_iota == r, p, acc)
        out_v[pl.ds(g * LANES, LANES)] = acc
        return carry

    lax.fori_loop(0, GROUPS, group_body, 0)

    pltpu.sync_copy(out_v, out_hbm.at[pl.ds(base, BPW)])


def kernel(user_ids, item_ids, user_table, item_table):
    return _cf_dot_kernel(user_ids.astype(jnp.int32),
                          item_ids.astype(jnp.int32),
                          user_table, item_table)


# speedup vs baseline: 1.5875x; 1.5875x over previous
import functools

import jax
import jax.numpy as jnp
from jax import lax
from jax.experimental import pallas as pl
from jax.experimental.pallas import tpu as pltpu
from jax.experimental.pallas import tpu_sc as plsc

BATCH = 16384
EMBED = 64
NUM_CORES = 2
NUM_SUBCORES = 16
NUM_WORKERS = NUM_CORES * NUM_SUBCORES
BPW = BATCH // NUM_WORKERS
LANES = 16
GROUPS = BPW // LANES
NCHUNKS = 4
CHUNK = BPW // NCHUNKS

_mesh = plsc.VectorSubcoreMesh(core_axis_name="c", subcore_axis_name="s")

_GATHER_DNUMS = lax.GatherDimensionNumbers(
    offset_dims=(), collapsed_slice_dims=(0,), start_index_map=(0,))


def _permute(x, pm):
    return lax.gather(x, pm[:, None], _GATHER_DNUMS, (1,),
                      mode=lax.GatherScatterMode.PROMISE_IN_BOUNDS)


@functools.partial(
    pl.kernel,
    mesh=_mesh,
    out_type=jax.ShapeDtypeStruct((BATCH,), jnp.float32),
    scratch_types=[
        pltpu.VMEM((BPW,), jnp.int32),
        pltpu.VMEM((BPW,), jnp.int32),
        pltpu.VMEM((CHUNK, EMBED), jnp.float32),
        pltpu.VMEM((CHUNK, EMBED), jnp.float32),
        pltpu.VMEM((CHUNK, EMBED), jnp.float32),
        pltpu.VMEM((CHUNK, EMBED), jnp.float32),
        pltpu.VMEM((BPW,), jnp.float32),
        pltpu.SemaphoreType.DMA,
        pltpu.SemaphoreType.DMA,
        pltpu.SemaphoreType.DMA,
        pltpu.SemaphoreType.DMA,
    ],
)
def _cf_dot_kernel(uid_hbm, iid_hbm, utab_hbm, itab_hbm, out_hbm,
                   uidx_v, iidx_v, urows0, urows1, irows0, irows1, out_v,
                   usem0, usem1, isem0, isem1):
    wid = lax.axis_index("s") * NUM_CORES + lax.axis_index("c")
    base = wid * BPW

    pltpu.sync_copy(uid_hbm.at[pl.ds(base, BPW)], uidx_v)
    pltpu.sync_copy(iid_hbm.at[pl.ds(base, BPW)], iidx_v)

    ubufs = (urows0, urows1)
    ibufs = (irows0, irows1)
    usems = (usem0, usem1)
    isems = (isem0, isem1)

    def fire(c):
        slot = c % 2

        def fire_body(g, carry):
            uvec = uidx_v[pl.ds(c * CHUNK + g * LANES, LANES)]
            ivec = iidx_v[pl.ds(c * CHUNK + g * LANES, LANES)]
            for r in range(LANES):
                b = g * LANES + r
                pltpu.async_copy(utab_hbm.at[uvec[r]], ubufs[slot].at[b],
                                 usems[slot])
                pltpu.async_copy(itab_hbm.at[ivec[r]], ibufs[slot].at[b],
                                 isems[slot])
            return carry

        lax.fori_loop(0, CHUNK // LANES, fire_body, 0)

    lane_iota = lax.iota(jnp.int32, LANES)
    perms = [(lane_iota + (1 << k)) & (LANES - 1) for k in range(4)]

    def compute(c):
        slot = c % 2

        def group_body(g, carry):
            acc = jnp.zeros((LANES,), jnp.float32)
            for r in range(LANES):
                b = g * LANES + r
                p = jnp.zeros((LANES,), jnp.float32)
                for k in range(EMBED // LANES):
                    sl = pl.ds(k * LANES, LANES)
                    p = p + ubufs[slot][b, sl] * ibufs[slot][b, sl]
                for pm in perms:
                    p = p + _permute(p, pm)
                acc = jnp.where(lane_iota == r, p, acc)
            out_v[pl.ds(c * CHUNK + g * LANES, LANES)] = acc
            return carry

        lax.fori_loop(0, CHUNK // LANES, group_body, 0)

    fire(0)
    fire(1)
    for c in range(NCHUNKS):
        slot = c % 2
        pltpu.make_async_copy(utab_hbm.at[pl.ds(0, CHUNK)], ubufs[slot],
                              usems[slot]).wait()
        pltpu.make_async_copy(itab_hbm.at[pl.ds(0, CHUNK)], ibufs[slot],
                              isems[slot]).wait()
        compute(c)
        if c + 2 < NCHUNKS:
            fire(c + 2)

    pltpu.sync_copy(out_v, out_hbm.at[pl.ds(base, BPW)])


def kernel(user_ids, item_ids, user_table, item_table):
    return _cf_dot_kernel(user_ids.astype(jnp.int32),
                          item_ids.astype(jnp.int32),
                          user_table, item_table)
